# baseline (speedup 1.0000x reference)
import functools

import jax
import jax.numpy as jnp
from jax import lax
from jax.experimental import pallas as pl
from jax.experimental.pallas import tpu as pltpu
from jax.experimental.pallas import tpu_sc as plsc

N = 10000
E = 320000
D = 128
DH = D // 2

NC = 2
NS = 16

CHUNK = 128
NBLK = E // CHUNK
BPT = NBLK // NS
XTRA = NBLK - BPT * NS
NB = 4
LOOK = 2
WB_TILES = 10
WB_ROWS = N // WB_TILES
ZROWS = 200


def _sc_aggregate(x2, eidx):
    mesh = plsc.VectorSubcoreMesh(core_axis_name="c", subcore_axis_name="s")

    @functools.partial(
        pl.kernel,
        mesh=mesh,
        out_type=jax.ShapeDtypeStruct((N, D), jnp.float32),
        scratch_types=[
            pltpu.VMEM((BPT + 1, 2, CHUNK), jnp.int32),
            pltpu.VMEM((NB, CHUNK, DH), jnp.float32),
            pltpu.VMEM((ZROWS, DH), jnp.float32),
            pltpu.VMEM_SHARED((N, DH), jnp.float32),
            [pltpu.SemaphoreType.DMA] * NB,
            [pltpu.SemaphoreType.DMA] * NB,
        ],
        compiler_params=pltpu.CompilerParams(use_tc_tiling_on_sc=False),
    )
    def agg_kernel(x_hbm, e_hbm, out_hbm, est, rows, zbuf, acc, gsem, ssem):
        cid = lax.axis_index("c")
        sid = lax.axis_index("s")

        pltpu.sync_copy(e_hbm.at[pl.ds(sid * BPT, BPT)],
                        est.at[pl.ds(0, BPT)])

        @pl.when(sid < XTRA)
        def _():
            pltpu.sync_copy(e_hbm.at[pl.ds(NS * BPT + sid, 1)],
                            est.at[pl.ds(BPT, 1)])

        def remap(c):
            def body(j, _):
                s = pl.ds(j * 16, 16)
                est[c, 0, s] = 2 * est[c, 0, s] + cid
                return 0
            lax.fori_loop(0, CHUNK // 16, body, 0)

        def wait_gather(b):
            pltpu.make_async_copy(x_hbm.at[pl.ds(0, CHUNK)], rows.at[b], gsem[b]).wait()

        def wait_scatter(b):
            pltpu.make_async_copy(x_hbm.at[pl.ds(0, CHUNK)], rows.at[b], ssem[b]).wait()

        def gather(c, b):
            pltpu.async_copy(x_hbm.at[est.at[c, 0]], rows.at[b], gsem[b])

        def scatter(c, b):
            pltpu.async_copy(rows.at[b], acc.at[est.at[c, 1]], ssem[b], add=True)

        for c in range(LOOK + 1):
            remap(c)
        for c in range(LOOK):
            gather(c, c)

        def zrow(i, _):
            for j in range(DH // 16):
                zbuf[i, pl.ds(j * 16, 16)] = jnp.zeros((16,), jnp.float32)
            return 0
        lax.fori_loop(0, ZROWS, zrow, 0)
        base = sid * WB_ROWS

        @pl.when(sid < WB_TILES)
        def _():
            for k in range(WB_ROWS // ZROWS):
                pltpu.sync_copy(zbuf, acc.at[pl.ds(base + k * ZROWS, ZROWS)])
        plsc.subcore_barrier()

        def group(g, _):
            for b in range(NB):
                c = g * NB + b
                b2 = (b + LOOK) % NB
                c2 = c + LOOK

                @pl.when(c + LOOK + 1 < BPT)
                def _():
                    remap(c + LOOK + 1)

                @pl.when(c2 < BPT)
                def _():
                    @pl.when(c >= NB - LOOK)
                    def _():
                        wait_scatter(b2)
                    gather(c2, b2)

                wait_gather(b)
                scatter(c, b)
            return 0

        lax.fori_loop(0, BPT // NB, group, 0)
        for b in range(NB):
            wait_scatter(b)

        @pl.when(sid < XTRA)
        def _():
            remap(BPT)
            gather(BPT, 0)
            wait_gather(0)
            scatter(BPT, 0)
            wait_scatter(0)

        plsc.subcore_barrier()

        @pl.when(sid < WB_TILES)
        def _():
            pltpu.sync_copy(acc.at[pl.ds(base, WB_ROWS)],
                            out_hbm.at[pl.ds(base, WB_ROWS),
                                       pl.ds(cid * DH, DH)])

    return agg_kernel(x2, eidx)


def _tc_mlp_block(scale_ref, x_ref, p_ref, w1_ref, b1_ref,
                  w2_ref, b2_ref, out_ref):
    t = scale_ref[0, 0] * x_ref[...] + p_ref[...]
    h = jnp.dot(t, w1_ref[...], preferred_element_type=jnp.float32) + b1_ref[...]
    h = jnp.maximum(h, 0.0)
    out_ref[...] = (
        jnp.dot(h, w2_ref[...], preferred_element_type=jnp.float32) + b2_ref[...]
    )


def _tc_mlp(x, agg, scale, W1, b1, W2, b2):
    rows = 2000
    grid = (N // rows,)
    return pl.pallas_call(
        _tc_mlp_block,
        grid=grid,
        in_specs=[
            pl.BlockSpec(memory_space=pltpu.SMEM),
            pl.BlockSpec((rows, D), lambda i: (i, 0)),
            pl.BlockSpec((rows, D), lambda i: (i, 0)),
            pl.BlockSpec((D, D), lambda i: (0, 0)),
            pl.BlockSpec((1, D), lambda i: (0, 0)),
            pl.BlockSpec((D, D), lambda i: (0, 0)),
            pl.BlockSpec((1, D), lambda i: (0, 0)),
        ],
        out_specs=pl.BlockSpec((rows, D), lambda i: (i, 0)),
        out_shape=jax.ShapeDtypeStruct((N, D), jnp.float32),
    )(scale, x, agg, W1, b1, W2, b2)


def kernel(x, edge_index, eps, W1, b1, W2, b2):
    ei = edge_index.astype(jnp.int32)
    eidx = ei.reshape(2, NBLK, CHUNK).transpose(1, 0, 2)
    agg = _sc_aggregate(x.reshape(2 * N, DH), eidx)
    scale = (1.0 + eps).astype(jnp.float32).reshape(1, 1)
    return _tc_mlp(x, agg, scale, W1.astype(jnp.float32),
                   b1.reshape(1, D), W2.astype(jnp.float32), b2.reshape(1, D))
